# baseline (speedup 1.0000x reference)
import functools

import jax
import jax.numpy as jnp
from jax import lax
from jax.experimental import pallas as pl
from jax.experimental.pallas import tpu as pltpu
from jax.experimental.pallas import tpu_sc as plsc

HDIM = 64
NC = 2
NS = 16
NW = NC * NS
HALF_A = 104
HALF_B = 96
NBUF = 8
PDIST = 4


def _gather_kernel(B, S):
    rows_w = B // NW

    mesh = plsc.VectorSubcoreMesh(
        core_axis_name="c", subcore_axis_name="s",
        num_cores=NC, num_subcores=NS,
    )

    @functools.partial(
        pl.kernel,
        out_type=jax.ShapeDtypeStruct((B * S, 2 * HDIM), jnp.float32),
        mesh=mesh,
        scratch_types=[
            pltpu.VMEM((rows_w, S), jnp.int32),
            [pltpu.VMEM((S, HDIM), jnp.float32) for _ in range(NBUF)],
            [pltpu.SemaphoreType.DMA for _ in range(NBUF)],
            [pltpu.SemaphoreType.DMA for _ in range(NBUF)],
        ],
        compiler_params=pltpu.CompilerParams(use_tc_tiling_on_sc=False),
    )
    def k(x_hbm, table_hbm, out_hbm, idx_v, bufs, gsems, osems):
        wid = lax.axis_index("s") * NC + lax.axis_index("c")
        row0 = wid * rows_w
        pltpu.sync_copy(x_hbm.at[pl.ds(row0, rows_w)], idx_v)

        def wb_dst(r):
            return out_hbm.at[pl.ds((row0 + r) * S, S), pl.ds(0, HDIM)]

        def fire(r, b):
            pltpu.async_copy(table_hbm.at[idx_v.at[r, pl.ds(0, HALF_A)]],
                             bufs[b].at[pl.ds(0, HALF_A)], gsems[b])
            pltpu.async_copy(table_hbm.at[idx_v.at[r, pl.ds(HALF_A, HALF_B)]],
                             bufs[b].at[pl.ds(HALF_A, HALF_B)], gsems[b])

        def wait_fire(b):
            pltpu.make_async_copy(table_hbm.at[idx_v.at[0, pl.ds(0, S)]],
                                  bufs[b], gsems[b]).wait()

        def wait_wb(r, b):
            pltpu.make_async_copy(bufs[b], wb_dst(r), osems[b]).wait()

        for b in range(PDIST):
            fire(b, b)

        def group(g, carry):
            for b in range(NBUF):
                r = g * NBUF + b
                rp = r + PDIST
                bp = (b + PDIST) % NBUF

                @pl.when(jnp.logical_and(rp >= NBUF, rp < rows_w))
                def _():
                    wait_wb(rp - NBUF, bp)

                @pl.when(rp < rows_w)
                def _():
                    fire(rp, bp)

                wait_fire(b)
                pltpu.async_copy(bufs[b], wb_dst(r), osems[b])
            return carry

        lax.fori_loop(0, rows_w // NBUF, group, 0)

        for b in range(NBUF):
            wait_wb(rows_w - NBUF + b, b)

    return k


def _transpose_kernel(V):
    NBLK = V // 128
    TAILV = V - NBLK * 128
    BLK_W = 128 * HDIM
    base_cnt, extra = NBLK // NW, NBLK % NW

    mesh = plsc.VectorSubcoreMesh(
        core_axis_name="c", subcore_axis_name="s",
        num_cores=NC, num_subcores=NS,
    )

    @functools.partial(
        pl.kernel,
        out_type=jax.ShapeDtypeStruct((V * HDIM,), jnp.float32),
        mesh=mesh,
        scratch_types=[
            [pltpu.VMEM((HDIM, 128), jnp.float32) for _ in range(2)],
            [pltpu.VMEM((BLK_W,), jnp.float32) for _ in range(2)],
            [pltpu.SemaphoreType.DMA for _ in range(2)],
            [pltpu.SemaphoreType.DMA for _ in range(2)],
        ],
        compiler_params=pltpu.CompilerParams(use_tc_tiling_on_sc=True,
                                             needs_layout_passes=False),
    )
    def k0(wt_hbm, tail_hbm, t1d_hbm, ins, outs, isems, osems):
        wid = lax.axis_index("s") * NC + lax.axis_index("c")
        cnt = base_cnt + (wid < extra).astype(jnp.int32)
        iota64 = lax.iota(jnp.int32, 16) * HDIM

        def blk(i):
            return wid + NW * i

        def start_in(i, p):
            pltpu.async_copy(wt_hbm.at[:, pl.ds(blk(i) * 128, 128)],
                             ins[p], isems[p])

        def wait_in(p):
            pltpu.make_async_copy(wt_hbm.at[:, pl.ds(0, 128)],
                                  ins[p], isems[p]).wait()

        def start_out(i, p):
            pltpu.async_copy(outs[p], t1d_hbm.at[pl.ds(blk(i) * BLK_W, BLK_W)],
                             osems[p])

        def wait_out(p):
            pltpu.make_async_copy(outs[p], t1d_hbm.at[pl.ds(0, BLK_W)],
                                  osems[p]).wait()

        def transpose(src, dst):
            def hbody(h, carry):
                for kk in range(8):
                    val = src[h, pl.ds(16 * kk, 16)]
                    idx = iota64 + (16 * kk * HDIM + h)
                    plsc.store_scatter(dst, [idx], val)
                return carry
            lax.fori_loop(0, HDIM, hbody, 0)

        start_in(0, 0)

        def grp(g, carry):
            for q in range(2):
                i = 2 * g + q

                @pl.when(i < cnt)
                def _():
                    wait_in(q)

                    @pl.when(i + 1 < cnt)
                    def _():
                        start_in(i + 1, 1 - q)

                    @pl.when(i >= 2)
                    def _():
                        wait_out(q)

                    transpose(ins[q], outs[q])
                    start_out(i, q)
            return carry

        lax.fori_loop(0, (base_cnt + 2) // 2, grp, 0)
        wait_out(0)
        wait_out(1)

        @pl.when(wid == NW - 1)
        def _():
            pltpu.sync_copy(tail_hbm, ins[0])
            transpose(ins[0], outs[0])
            pltpu.sync_copy(outs[0].at[pl.ds(0, TAILV * HDIM)],
                            t1d_hbm.at[pl.ds(NBLK * BLK_W, TAILV * HDIM)])

    return k0


def kernel(x, weight):
    B, S = x.shape
    V = weight.shape[0]
    n_tail = V % 128
    wt = weight.T
    tail = jnp.pad(weight[V - n_tail:].T, ((0, 0), (0, 128 - n_tail)))
    t1d = _transpose_kernel(V)(wt, tail)
    table = t1d.reshape(V, HDIM)
    out128 = _gather_kernel(B, S)(x, table)
    return out128[:, :HDIM].reshape(B, S, HDIM)
